# baseline (device time: 607270 ns/iter reference)
import jax
import jax.numpy as jnp
from jax import lax
from jax.experimental import pallas as pl
from jax.experimental.pallas import tpu as pltpu

CHUNK = 128

BM, BN, BK = 512, 1024, 2048


def _matmul_bf16(x, W):
    T, D = x.shape
    _, V = W.shape

    def mm_body(x_ref, w_ref, o_ref, acc_ref):
        k = pl.program_id(2)

        @pl.when(k == 0)
        def _():
            acc_ref[...] = jnp.zeros_like(acc_ref)

        acc_ref[...] += jnp.dot(
            x_ref[...].astype(jnp.bfloat16),
            w_ref[...].astype(jnp.bfloat16),
            preferred_element_type=jnp.float32,
        )

        @pl.when(k == D // BK - 1)
        def _():
            o_ref[...] = acc_ref[...].astype(jnp.bfloat16)

    return pl.pallas_call(
        mm_body,
        grid=(T // BM, V // BN, D // BK),
        in_specs=[
            pl.BlockSpec((BM, BK), lambda m, n, k: (m, k)),
            pl.BlockSpec((BK, BN), lambda m, n, k: (k, n)),
        ],
        out_specs=pl.BlockSpec((BM, BN), lambda m, n, k: (m, n)),
        out_shape=jax.ShapeDtypeStruct((T, V), jnp.bfloat16),
        scratch_shapes=[pltpu.VMEM((BM, BN), jnp.float32)],
        compiler_params=pltpu.CompilerParams(
            dimension_semantics=("parallel", "parallel", "arbitrary"),
            vmem_limit_bytes=56 * 1024 * 1024,
        ),
    )(x, W)


def kernel(x, W):
    T, D = x.shape
    _, V = W.shape
    n_chunks = T // CHUNK
    K = n_chunks // 2

    logits = _matmul_bf16(x, W)

    def body(lg0_ref, lg1_ref, outa_ref, outb_ref,
             loc_ref, othr_ref, xr_ref, yr_ref,
             xsend_sems, xrecv_sems, ysend_sems, yrecv_sems,
             xcredit, ycredit):
        q = pl.program_id(0)
        my_x = lax.axis_index("x")
        my_y = lax.axis_index("y")
        xn = (1 - my_x, my_y)
        yn = (my_x, 1 - my_y)
        s2 = lax.rem(q, 2)
        s4 = lax.rem(q, 4)
        p2 = lax.rem(q + 1, 2)
        p4 = lax.rem(q + 3, 4)

        def xdesc(slot4, slot2):
            return pltpu.make_async_remote_copy(
                src_ref=loc_ref.at[slot2],
                dst_ref=xr_ref.at[slot4],
                send_sem=xsend_sems.at[slot2],
                recv_sem=xrecv_sems.at[slot4],
                device_id=xn,
                device_id_type=pl.DeviceIdType.MESH,
            )

        def ydesc(slot4):
            return pltpu.make_async_remote_copy(
                src_ref=xr_ref.at[slot4],
                dst_ref=yr_ref.at[slot4],
                send_sem=ysend_sems.at[slot4],
                recv_sem=yrecv_sems.at[slot4],
                device_id=yn,
                device_id_type=pl.DeviceIdType.MESH,
            )

        def softmax_store(out_ref, lloc_bf16, lrem_bf16):
            eloc = jnp.exp(lloc_bf16)
            erem = jnp.exp(lrem_bf16)
            denom = (
                jnp.sum(eloc, axis=-1, keepdims=True, dtype=jnp.float32)
                + jnp.sum(erem, axis=-1, keepdims=True, dtype=jnp.float32)
            )
            r = 1.0 / denom
            out_ref[:, pl.ds(my_x * V, V)] = (
                eloc.astype(jnp.float32) * r
            ).astype(jnp.bfloat16)
            out_ref[:, pl.ds((1 - my_x) * V, V)] = (
                erem.astype(jnp.float32) * r
            ).astype(jnp.bfloat16)

        @pl.when(q == 0)
        def _():
            bar = pltpu.get_barrier_semaphore()
            for nbr in (xn, yn):
                pl.semaphore_signal(
                    bar, inc=1, device_id=nbr,
                    device_id_type=pl.DeviceIdType.MESH,
                )
            pl.semaphore_wait(bar, 2)

        @pl.when(q < K)
        def _():
            @pl.when(q >= 2)
            def _():
                xdesc(s4, s2).wait_send()
            @pl.when(q >= 4)
            def _():
                pl.semaphore_wait(xcredit, 1)

            @pl.when(my_y == 0)
            def _():
                loc_ref[s2] = lg0_ref[...]
                othr_ref[s2] = lg1_ref[...]
            @pl.when(my_y == 1)
            def _():
                loc_ref[s2] = lg1_ref[...]
                othr_ref[s2] = lg0_ref[...]

            xdesc(s4, s2).start()
            xdesc(s4, s2).wait_recv()

            @pl.when(q >= 2)
            def _():
                ydesc(lax.rem(q + 2, 4)).wait_send()
                @pl.when(q <= 5)
                def _():
                    pl.semaphore_signal(
                        xcredit, inc=1, device_id=xn,
                        device_id_type=pl.DeviceIdType.MESH,
                    )
            @pl.when(q >= 4)
            def _():
                pl.semaphore_wait(ycredit, 1)
            ydesc(s4).start()

            softmax_store(outa_ref, loc_ref[s2], xr_ref[s4])

        @pl.when(jnp.logical_and(q >= 1, q <= K))
        def _():
            ydesc(p4).wait_recv()
            softmax_store(outb_ref, othr_ref[p2], yr_ref[p4])
            @pl.when(q <= 4)
            def _():
                pl.semaphore_signal(
                    ycredit, inc=1, device_id=yn,
                    device_id_type=pl.DeviceIdType.MESH,
                )

        @pl.when(q == K + 1)
        def _():
            xdesc(0, 0).wait_send()
            xdesc(0, 1).wait_send()
            ydesc(2).wait_send()
            ydesc(3).wait_send()

    qa = lambda q: (jnp.minimum(q, K - 1), 0)
    qb = lambda q: (jnp.clip(q - 1, 0, K - 1), 0)

    outa, outb = pl.pallas_call(
        body,
        grid=(K + 2,),
        in_specs=[
            pl.BlockSpec((CHUNK, V), qa),
            pl.BlockSpec((CHUNK, V), lambda q: (K + jnp.minimum(q, K - 1), 0)),
        ],
        out_specs=[
            pl.BlockSpec((CHUNK, 2 * V), qa),
            pl.BlockSpec((CHUNK, 2 * V), qb),
        ],
        out_shape=[
            jax.ShapeDtypeStruct((K * CHUNK, 2 * V), jnp.bfloat16),
            jax.ShapeDtypeStruct((K * CHUNK, 2 * V), jnp.bfloat16),
        ],
        scratch_shapes=[
            pltpu.VMEM((2, CHUNK, V), jnp.bfloat16),
            pltpu.VMEM((2, CHUNK, V), jnp.bfloat16),
            pltpu.VMEM((4, CHUNK, V), jnp.bfloat16),
            pltpu.VMEM((4, CHUNK, V), jnp.bfloat16),
            pltpu.SemaphoreType.DMA((2,)),
            pltpu.SemaphoreType.DMA((4,)),
            pltpu.SemaphoreType.DMA((4,)),
            pltpu.SemaphoreType.DMA((4,)),
            pltpu.SemaphoreType.REGULAR,
            pltpu.SemaphoreType.REGULAR,
        ],
        compiler_params=pltpu.CompilerParams(
            collective_id=0,
            vmem_limit_bytes=60 * 1024 * 1024,
        ),
    )(logits, logits)

    my_y = lax.axis_index("y")
    return lax.cond(
        my_y == 0,
        lambda: jnp.concatenate([outa, outb], axis=0),
        lambda: jnp.concatenate([outb, outa], axis=0),
    )


# device time: 537122 ns/iter; 1.1306x vs baseline; 1.1306x over previous
import jax
import jax.numpy as jnp
from jax import lax
from jax.experimental import pallas as pl
from jax.experimental.pallas import tpu as pltpu

CHUNK = 128

BM, BN, BK = 2048, 1024, 512


def _matmul_bf16(x, W):
    T, D = x.shape
    _, V = W.shape

    def mm_body(x_ref, w_ref, o_ref, acc_ref):
        k = pl.program_id(2)

        @pl.when(k == 0)
        def _():
            acc_ref[...] = jnp.zeros_like(acc_ref)

        acc_ref[...] += jnp.dot(
            x_ref[...].astype(jnp.bfloat16),
            w_ref[...].astype(jnp.bfloat16),
            preferred_element_type=jnp.float32,
        )

        @pl.when(k == D // BK - 1)
        def _():
            o_ref[...] = acc_ref[...].astype(jnp.bfloat16)

    return pl.pallas_call(
        mm_body,
        grid=(T // BM, V // BN, D // BK),
        in_specs=[
            pl.BlockSpec((BM, BK), lambda m, n, k: (m, k)),
            pl.BlockSpec((BK, BN), lambda m, n, k: (k, n)),
        ],
        out_specs=pl.BlockSpec((BM, BN), lambda m, n, k: (m, n)),
        out_shape=jax.ShapeDtypeStruct((T, V), jnp.bfloat16),
        scratch_shapes=[pltpu.VMEM((BM, BN), jnp.float32)],
        compiler_params=pltpu.CompilerParams(
            dimension_semantics=("parallel", "parallel", "arbitrary"),
            vmem_limit_bytes=56 * 1024 * 1024,
        ),
    )(x, W)


def kernel(x, W):
    T, D = x.shape
    _, V = W.shape
    n_chunks = T // CHUNK
    K = n_chunks // 2

    logits = _matmul_bf16(x, W)

    def body(lg0_ref, lg1_ref, outa_ref, outb_ref,
             loc_ref, othr_ref, xr_ref, yr_ref,
             xsend_sems, xrecv_sems, ysend_sems, yrecv_sems,
             xcredit, ycredit):
        q = pl.program_id(0)
        my_x = lax.axis_index("x")
        my_y = lax.axis_index("y")
        xn = (1 - my_x, my_y)
        yn = (my_x, 1 - my_y)
        s2 = lax.rem(q, 2)
        s4 = lax.rem(q, 4)
        p2 = lax.rem(q + 1, 2)
        p4 = lax.rem(q + 3, 4)

        def xdesc(slot4, slot2):
            return pltpu.make_async_remote_copy(
                src_ref=loc_ref.at[slot2],
                dst_ref=xr_ref.at[slot4],
                send_sem=xsend_sems.at[slot2],
                recv_sem=xrecv_sems.at[slot4],
                device_id=xn,
                device_id_type=pl.DeviceIdType.MESH,
            )

        def ydesc(slot4):
            return pltpu.make_async_remote_copy(
                src_ref=xr_ref.at[slot4],
                dst_ref=yr_ref.at[slot4],
                send_sem=ysend_sems.at[slot4],
                recv_sem=yrecv_sems.at[slot4],
                device_id=yn,
                device_id_type=pl.DeviceIdType.MESH,
            )

        def softmax_store(out_ref, lloc_bf16, lrem_bf16):
            eloc = jnp.exp(lloc_bf16)
            erem = jnp.exp(lrem_bf16)
            denom = (
                jnp.sum(eloc, axis=-1, keepdims=True, dtype=jnp.float32)
                + jnp.sum(erem, axis=-1, keepdims=True, dtype=jnp.float32)
            )
            r = 1.0 / denom
            out_ref[:, pl.ds(my_x * V, V)] = (
                eloc.astype(jnp.float32) * r
            ).astype(jnp.bfloat16)
            out_ref[:, pl.ds((1 - my_x) * V, V)] = (
                erem.astype(jnp.float32) * r
            ).astype(jnp.bfloat16)

        @pl.when(q == 0)
        def _():
            bar = pltpu.get_barrier_semaphore()
            for nbr in (xn, yn):
                pl.semaphore_signal(
                    bar, inc=1, device_id=nbr,
                    device_id_type=pl.DeviceIdType.MESH,
                )
            pl.semaphore_wait(bar, 2)

        @pl.when(q < K)
        def _():
            @pl.when(q >= 2)
            def _():
                xdesc(s4, s2).wait_send()
            @pl.when(q >= 4)
            def _():
                pl.semaphore_wait(xcredit, 1)

            @pl.when(my_y == 0)
            def _():
                loc_ref[s2] = lg0_ref[...]
                othr_ref[s2] = lg1_ref[...]
            @pl.when(my_y == 1)
            def _():
                loc_ref[s2] = lg1_ref[...]
                othr_ref[s2] = lg0_ref[...]

            xdesc(s4, s2).start()
            xdesc(s4, s2).wait_recv()

            @pl.when(q >= 2)
            def _():
                ydesc(lax.rem(q + 2, 4)).wait_send()
                @pl.when(q <= 5)
                def _():
                    pl.semaphore_signal(
                        xcredit, inc=1, device_id=xn,
                        device_id_type=pl.DeviceIdType.MESH,
                    )
            @pl.when(q >= 4)
            def _():
                pl.semaphore_wait(ycredit, 1)
            ydesc(s4).start()

            softmax_store(outa_ref, loc_ref[s2], xr_ref[s4])

        @pl.when(jnp.logical_and(q >= 1, q <= K))
        def _():
            ydesc(p4).wait_recv()
            softmax_store(outb_ref, othr_ref[p2], yr_ref[p4])
            @pl.when(q <= 4)
            def _():
                pl.semaphore_signal(
                    ycredit, inc=1, device_id=yn,
                    device_id_type=pl.DeviceIdType.MESH,
                )

        @pl.when(q == K + 1)
        def _():
            xdesc(0, 0).wait_send()
            xdesc(0, 1).wait_send()
            ydesc(2).wait_send()
            ydesc(3).wait_send()

    qa = lambda q: (jnp.minimum(q, K - 1), 0)
    qb = lambda q: (jnp.clip(q - 1, 0, K - 1), 0)

    outa, outb = pl.pallas_call(
        body,
        grid=(K + 2,),
        in_specs=[
            pl.BlockSpec((CHUNK, V), qa),
            pl.BlockSpec((CHUNK, V), lambda q: (K + jnp.minimum(q, K - 1), 0)),
        ],
        out_specs=[
            pl.BlockSpec((CHUNK, 2 * V), qa),
            pl.BlockSpec((CHUNK, 2 * V), qb),
        ],
        out_shape=[
            jax.ShapeDtypeStruct((K * CHUNK, 2 * V), jnp.bfloat16),
            jax.ShapeDtypeStruct((K * CHUNK, 2 * V), jnp.bfloat16),
        ],
        scratch_shapes=[
            pltpu.VMEM((2, CHUNK, V), jnp.bfloat16),
            pltpu.VMEM((2, CHUNK, V), jnp.bfloat16),
            pltpu.VMEM((4, CHUNK, V), jnp.bfloat16),
            pltpu.VMEM((4, CHUNK, V), jnp.bfloat16),
            pltpu.SemaphoreType.DMA((2,)),
            pltpu.SemaphoreType.DMA((4,)),
            pltpu.SemaphoreType.DMA((4,)),
            pltpu.SemaphoreType.DMA((4,)),
            pltpu.SemaphoreType.REGULAR,
            pltpu.SemaphoreType.REGULAR,
        ],
        compiler_params=pltpu.CompilerParams(
            collective_id=0,
            vmem_limit_bytes=60 * 1024 * 1024,
        ),
    )(logits, logits)

    my_y = lax.axis_index("y")
    return lax.cond(
        my_y == 0,
        lambda: jnp.concatenate([outa, outb], axis=0),
        lambda: jnp.concatenate([outb, outa], axis=0),
    )


# device time: 493483 ns/iter; 1.2306x vs baseline; 1.0884x over previous
import jax
import jax.numpy as jnp
from jax import lax
from jax.experimental import pallas as pl
from jax.experimental.pallas import tpu as pltpu

CHUNK = 128

BM, BN, BK = 2048, 1024, 512


def _matmul_bf16(x, W):
    T, D = x.shape
    _, V = W.shape

    def mm_body(x_ref, w_ref, o_ref, acc_ref):
        k = pl.program_id(2)

        @pl.when(k == 0)
        def _():
            acc_ref[...] = jnp.zeros_like(acc_ref)

        acc_ref[...] += jnp.dot(
            x_ref[...].astype(jnp.bfloat16),
            w_ref[...].astype(jnp.bfloat16),
            preferred_element_type=jnp.float32,
        )

        @pl.when(k == D // BK - 1)
        def _():
            o_ref[...] = acc_ref[...].astype(jnp.bfloat16)

    return pl.pallas_call(
        mm_body,
        grid=(T // BM, V // BN, D // BK),
        in_specs=[
            pl.BlockSpec((BM, BK), lambda m, n, k: (m, k)),
            pl.BlockSpec((BK, BN), lambda m, n, k: (k, n)),
        ],
        out_specs=pl.BlockSpec((BM, BN), lambda m, n, k: (m, n)),
        out_shape=jax.ShapeDtypeStruct((T, V), jnp.bfloat16),
        scratch_shapes=[pltpu.VMEM((BM, BN), jnp.float32)],
        compiler_params=pltpu.CompilerParams(
            dimension_semantics=("parallel", "parallel", "arbitrary"),
            vmem_limit_bytes=56 * 1024 * 1024,
        ),
    )(x, W)


def kernel(x, W):
    T, D = x.shape
    _, V = W.shape
    n_chunks = T // CHUNK
    K = n_chunks // 2

    logits = _matmul_bf16(x, W)

    def body(lg0_ref, lg1_ref, out_ref,
             loc_ref, othr_ref, xr_ref, yr_ref, stage_ref,
             xsend_sems, xrecv_sems, ysend_sems, yrecv_sems,
             stage_sems, xcredit, ycredit):
        q = pl.program_id(0)
        my_x = lax.axis_index("x")
        my_y = lax.axis_index("y")
        xn = (1 - my_x, my_y)
        yn = (my_x, 1 - my_y)
        s2 = lax.rem(q, 2)
        s4 = lax.rem(q, 4)
        p2 = lax.rem(q + 1, 2)
        p4 = lax.rem(q + 3, 4)

        def xdesc(slot4, slot2):
            return pltpu.make_async_remote_copy(
                src_ref=loc_ref.at[slot2],
                dst_ref=xr_ref.at[slot4],
                send_sem=xsend_sems.at[slot2],
                recv_sem=xrecv_sems.at[slot4],
                device_id=xn,
                device_id_type=pl.DeviceIdType.MESH,
            )

        def ydesc(slot4):
            return pltpu.make_async_remote_copy(
                src_ref=xr_ref.at[slot4],
                dst_ref=yr_ref.at[slot4],
                send_sem=ysend_sems.at[slot4],
                recv_sem=yrecv_sems.at[slot4],
                device_id=yn,
                device_id_type=pl.DeviceIdType.MESH,
            )

        def softmax_store(slot, chunk_id, lloc_bf16, lrem_bf16):
            eloc = jnp.exp(lloc_bf16)
            erem = jnp.exp(lrem_bf16)
            denom = (
                jnp.sum(eloc, axis=-1, keepdims=True, dtype=jnp.float32)
                + jnp.sum(erem, axis=-1, keepdims=True, dtype=jnp.float32)
            )
            r = 1.0 / denom
            stage_ref[slot, :, pl.ds(my_x * V, V)] = (
                eloc.astype(jnp.float32) * r
            ).astype(jnp.bfloat16)
            stage_ref[slot, :, pl.ds((1 - my_x) * V, V)] = (
                erem.astype(jnp.float32) * r
            ).astype(jnp.bfloat16)
            pltpu.make_async_copy(
                stage_ref.at[slot],
                out_ref.at[pl.ds(chunk_id * CHUNK, CHUNK), :],
                stage_sems.at[slot],
            ).start()

        def stage_wait(slot):
            pltpu.make_async_copy(
                stage_ref.at[slot],
                out_ref.at[pl.ds(0, CHUNK), :],
                stage_sems.at[slot],
            ).wait()

        @pl.when(q == 0)
        def _():
            bar = pltpu.get_barrier_semaphore()
            for nbr in (xn, yn):
                pl.semaphore_signal(
                    bar, inc=1, device_id=nbr,
                    device_id_type=pl.DeviceIdType.MESH,
                )
            pl.semaphore_wait(bar, 2)

        @pl.when(q < K)
        def _():
            @pl.when(q >= 2)
            def _():
                xdesc(s4, s2).wait_send()
            @pl.when(q >= 4)
            def _():
                pl.semaphore_wait(xcredit, 1)

            @pl.when(my_y == 0)
            def _():
                loc_ref[s2] = lg0_ref[...]
                othr_ref[s2] = lg1_ref[...]
            @pl.when(my_y == 1)
            def _():
                loc_ref[s2] = lg1_ref[...]
                othr_ref[s2] = lg0_ref[...]

            xdesc(s4, s2).start()
            xdesc(s4, s2).wait_recv()

            @pl.when(q >= 2)
            def _():
                ydesc(lax.rem(q + 2, 4)).wait_send()
                @pl.when(q <= 5)
                def _():
                    pl.semaphore_signal(
                        xcredit, inc=1, device_id=xn,
                        device_id_type=pl.DeviceIdType.MESH,
                    )
            @pl.when(q >= 4)
            def _():
                pl.semaphore_wait(ycredit, 1)
            ydesc(s4).start()

            @pl.when(q >= 2)
            def _():
                stage_wait(s2)
            softmax_store(s2, q + K * my_y, loc_ref[s2], xr_ref[s4])

        @pl.when(jnp.logical_and(q >= 1, q <= K))
        def _():
            ydesc(p4).wait_recv()
            @pl.when(q >= 3)
            def _():
                stage_wait(2 + p2)
            softmax_store(
                2 + p2, (q - 1) + K * (1 - my_y),
                othr_ref[p2], yr_ref[p4],
            )
            @pl.when(q <= 4)
            def _():
                pl.semaphore_signal(
                    ycredit, inc=1, device_id=yn,
                    device_id_type=pl.DeviceIdType.MESH,
                )

        @pl.when(q == K + 1)
        def _():
            xdesc(0, 0).wait_send()
            xdesc(0, 1).wait_send()
            ydesc(2).wait_send()
            ydesc(3).wait_send()
            stage_wait(0)
            stage_wait(1)
            stage_wait(2)
            stage_wait(3)

    qa = lambda q: (jnp.minimum(q, K - 1), 0)

    return pl.pallas_call(
        body,
        grid=(K + 2,),
        in_specs=[
            pl.BlockSpec((CHUNK, V), qa),
            pl.BlockSpec((CHUNK, V), lambda q: (K + jnp.minimum(q, K - 1), 0)),
        ],
        out_specs=pl.BlockSpec(memory_space=pltpu.MemorySpace.HBM),
        out_shape=jax.ShapeDtypeStruct((T, 2 * V), jnp.bfloat16),
        scratch_shapes=[
            pltpu.VMEM((2, CHUNK, V), jnp.bfloat16),
            pltpu.VMEM((2, CHUNK, V), jnp.bfloat16),
            pltpu.VMEM((4, CHUNK, V), jnp.bfloat16),
            pltpu.VMEM((4, CHUNK, V), jnp.bfloat16),
            pltpu.VMEM((4, CHUNK, 2 * V), jnp.bfloat16),
            pltpu.SemaphoreType.DMA((2,)),
            pltpu.SemaphoreType.DMA((4,)),
            pltpu.SemaphoreType.DMA((4,)),
            pltpu.SemaphoreType.DMA((4,)),
            pltpu.SemaphoreType.DMA((4,)),
            pltpu.SemaphoreType.REGULAR,
            pltpu.SemaphoreType.REGULAR,
        ],
        compiler_params=pltpu.CompilerParams(
            collective_id=0,
            vmem_limit_bytes=60 * 1024 * 1024,
        ),
    )(logits, logits)


# device time: 480451 ns/iter; 1.2640x vs baseline; 1.0271x over previous
import jax
import jax.numpy as jnp
from jax import lax
from jax.experimental import pallas as pl
from jax.experimental.pallas import tpu as pltpu

CHUNK = 128

BM, BN, BK = 2048, 1024, 512


def _matmul_bf16(x, W):
    T, D = x.shape
    _, V = W.shape

    def mm_body(x_ref, w_ref, o_ref, acc_ref):
        k = pl.program_id(2)

        @pl.when(k == 0)
        def _():
            acc_ref[...] = jnp.zeros_like(acc_ref)

        acc_ref[...] += jnp.dot(
            x_ref[...].astype(jnp.bfloat16),
            w_ref[...].astype(jnp.bfloat16),
            preferred_element_type=jnp.float32,
        )

        @pl.when(k == D // BK - 1)
        def _():
            o_ref[...] = acc_ref[...].astype(jnp.bfloat16)

    return pl.pallas_call(
        mm_body,
        grid=(T // BM, V // BN, D // BK),
        in_specs=[
            pl.BlockSpec((BM, BK), lambda m, n, k: (m, k)),
            pl.BlockSpec((BK, BN), lambda m, n, k: (k, n)),
        ],
        out_specs=pl.BlockSpec((BM, BN), lambda m, n, k: (m, n)),
        out_shape=jax.ShapeDtypeStruct((T, V), jnp.bfloat16),
        scratch_shapes=[pltpu.VMEM((BM, BN), jnp.float32)],
        compiler_params=pltpu.CompilerParams(
            dimension_semantics=("parallel", "parallel", "arbitrary"),
            vmem_limit_bytes=56 * 1024 * 1024,
        ),
    )(x, W)


def kernel(x, W):
    T, D = x.shape
    _, V = W.shape
    n_chunks = T // CHUNK
    K = n_chunks // 2

    logits = _matmul_bf16(x, W)

    def body(lg_ref, out_ref,
             loc_ref, othr_ref, xr_ref, yr_ref, stage_ref,
             xsend_sems, xrecv_sems, ysend_sems, yrecv_sems,
             stage_sems, xcredit, ycredit):
        q = pl.program_id(0)
        my_x = lax.axis_index("x")
        my_y = lax.axis_index("y")
        xn = (1 - my_x, my_y)
        yn = (my_x, 1 - my_y)
        s2 = lax.rem(q, 2)
        s4 = lax.rem(q, 4)
        p2 = lax.rem(q + 1, 2)
        p4 = lax.rem(q + 3, 4)

        def xdesc(slot4, slot2):
            return pltpu.make_async_remote_copy(
                src_ref=loc_ref.at[slot2],
                dst_ref=xr_ref.at[slot4],
                send_sem=xsend_sems.at[slot2],
                recv_sem=xrecv_sems.at[slot4],
                device_id=xn,
                device_id_type=pl.DeviceIdType.MESH,
            )

        def ydesc(slot4):
            return pltpu.make_async_remote_copy(
                src_ref=xr_ref.at[slot4],
                dst_ref=yr_ref.at[slot4],
                send_sem=ysend_sems.at[slot4],
                recv_sem=yrecv_sems.at[slot4],
                device_id=yn,
                device_id_type=pl.DeviceIdType.MESH,
            )

        def softmax_store(slot, chunk_id, lloc_bf16, lrem_bf16):
            eloc = jnp.exp(lloc_bf16)
            erem = jnp.exp(lrem_bf16)
            denom = (
                jnp.sum(eloc, axis=-1, keepdims=True, dtype=jnp.float32)
                + jnp.sum(erem, axis=-1, keepdims=True, dtype=jnp.float32)
            )
            r = (1.0 / denom).astype(jnp.bfloat16)
            stage_ref[slot, :, pl.ds(my_x * V, V)] = eloc * r
            stage_ref[slot, :, pl.ds((1 - my_x) * V, V)] = erem * r
            pltpu.make_async_copy(
                stage_ref.at[slot],
                out_ref.at[pl.ds(chunk_id * CHUNK, CHUNK), :],
                stage_sems.at[slot],
            ).start()

        def stage_wait(slot):
            pltpu.make_async_copy(
                stage_ref.at[slot],
                out_ref.at[pl.ds(0, CHUNK), :],
                stage_sems.at[slot],
            ).wait()

        @pl.when(q == 0)
        def _():
            bar = pltpu.get_barrier_semaphore()
            for nbr in (xn, yn):
                pl.semaphore_signal(
                    bar, inc=1, device_id=nbr,
                    device_id_type=pl.DeviceIdType.MESH,
                )
            pl.semaphore_wait(bar, 2)

        @pl.when(q < K)
        def _():
            @pl.when(q >= 2)
            def _():
                xdesc(s4, s2).wait_send()
            @pl.when(q >= 4)
            def _():
                pl.semaphore_wait(xcredit, 1)

            @pl.when(my_y == 0)
            def _():
                loc_ref[s2] = lg_ref[0]
                othr_ref[s2] = lg_ref[1]
            @pl.when(my_y == 1)
            def _():
                loc_ref[s2] = lg_ref[1]
                othr_ref[s2] = lg_ref[0]

            xdesc(s4, s2).start()
            xdesc(s4, s2).wait_recv()

            @pl.when(q >= 2)
            def _():
                ydesc(lax.rem(q + 2, 4)).wait_send()
                @pl.when(q <= 5)
                def _():
                    pl.semaphore_signal(
                        xcredit, inc=1, device_id=xn,
                        device_id_type=pl.DeviceIdType.MESH,
                    )
            @pl.when(q >= 4)
            def _():
                pl.semaphore_wait(ycredit, 1)
            ydesc(s4).start()

            @pl.when(q >= 2)
            def _():
                stage_wait(s2)
            softmax_store(s2, q + K * my_y, loc_ref[s2], xr_ref[s4])

        @pl.when(jnp.logical_and(q >= 1, q <= K))
        def _():
            ydesc(p4).wait_recv()
            @pl.when(q >= 3)
            def _():
                stage_wait(2 + p2)
            softmax_store(
                2 + p2, (q - 1) + K * (1 - my_y),
                othr_ref[p2], yr_ref[p4],
            )
            @pl.when(q <= 4)
            def _():
                pl.semaphore_signal(
                    ycredit, inc=1, device_id=yn,
                    device_id_type=pl.DeviceIdType.MESH,
                )

        @pl.when(q == K + 1)
        def _():
            xdesc(0, 0).wait_send()
            xdesc(0, 1).wait_send()
            ydesc(2).wait_send()
            ydesc(3).wait_send()
            stage_wait(0)
            stage_wait(1)
            stage_wait(2)
            stage_wait(3)

    return pl.pallas_call(
        body,
        grid=(K + 2,),
        in_specs=[
            pl.BlockSpec(
                (2, CHUNK, V),
                lambda q: (0, jnp.minimum(q, K - 1), 0),
            ),
        ],
        out_specs=pl.BlockSpec(memory_space=pltpu.MemorySpace.HBM),
        out_shape=jax.ShapeDtypeStruct((T, 2 * V), jnp.bfloat16),
        scratch_shapes=[
            pltpu.VMEM((2, CHUNK, V), jnp.bfloat16),
            pltpu.VMEM((2, CHUNK, V), jnp.bfloat16),
            pltpu.VMEM((4, CHUNK, V), jnp.bfloat16),
            pltpu.VMEM((4, CHUNK, V), jnp.bfloat16),
            pltpu.VMEM((4, CHUNK, 2 * V), jnp.bfloat16),
            pltpu.SemaphoreType.DMA((2,)),
            pltpu.SemaphoreType.DMA((4,)),
            pltpu.SemaphoreType.DMA((4,)),
            pltpu.SemaphoreType.DMA((4,)),
            pltpu.SemaphoreType.DMA((4,)),
            pltpu.SemaphoreType.REGULAR,
            pltpu.SemaphoreType.REGULAR,
        ],
        compiler_params=pltpu.CompilerParams(
            collective_id=0,
            vmem_limit_bytes=60 * 1024 * 1024,
        ),
    )(logits.reshape(2, K * CHUNK, V))
